# Initial kernel scaffold; baseline (speedup 1.0000x reference)
import jax
import jax.numpy as jnp
from jax.experimental import pallas as pl


def kernel(in_degree, out_degree, W_in, W_out):
    raise NotImplementedError("write your pallas kernel here")


# speedup vs baseline: 4.3394x; 4.3394x over previous
import functools

import jax
import jax.numpy as jnp
from jax import lax
from jax.experimental import pallas as pl
from jax.experimental.pallas import tpu as pltpu
from jax.experimental.pallas import tpu_sc as plsc

MAX_DEG = 64
VOCAB = MAX_DEG + 1
D = 128
N_ROWS = 100000
NUM_CORES = 2
NUM_SUBCORES = 16
NW = NUM_CORES * NUM_SUBCORES
CH = 128
NFULL = N_ROWS // CH
TAIL = N_ROWS - NFULL * CH
NCH = NFULL + 1
PAD_N = NCH * CH
CPW = (NCH + NW - 1) // NW


def _prep_body(win_ref, wout_ref, ind_ref, outd_ref, wsum_ref, idx_ref):
    win = win_ref[...]
    wout = wout_ref[...]
    wsum_ref[...] = win[:, None, :] + wout[None, :, :]
    a = jnp.clip(ind_ref[...], 0, MAX_DEG)
    b = jnp.clip(outd_ref[...], 0, MAX_DEG)
    idx_ref[...] = a * VOCAB + b


_mesh = plsc.VectorSubcoreMesh(core_axis_name="c", subcore_axis_name="s")


@functools.partial(
    pl.kernel,
    mesh=_mesh,
    out_type=jax.ShapeDtypeStruct((N_ROWS, D), jnp.float32),
    scratch_types=[
        pltpu.VMEM((CH,), jnp.int32),
        pltpu.VMEM((CH, D), jnp.float32),
        pltpu.SemaphoreType.DMA,
    ],
)
def _sc_gather(wsum_hbm, idx_hbm, out_hbm, idx_v, rows_v, sem):
    wid = lax.axis_index("s") * NUM_CORES + lax.axis_index("c")
    for c in range(CPW):
        g = c * NW + wid

        @pl.when(g < NCH)
        def _():
            pltpu.sync_copy(idx_hbm.at[pl.ds(g * CH, CH)], idx_v)
            pltpu.async_copy(wsum_hbm.at[idx_v], rows_v, sem).wait()

        @pl.when(g < NFULL)
        def _():
            pltpu.sync_copy(rows_v, out_hbm.at[pl.ds(g * CH, CH)])

        @pl.when(g == NFULL)
        def _():
            pltpu.sync_copy(rows_v.at[pl.ds(0, TAIL)],
                            out_hbm.at[pl.ds(g * CH, TAIL)])


def kernel(in_degree, out_degree, W_in, W_out):
    pad = PAD_N - N_ROWS
    ind = jnp.pad(in_degree.astype(jnp.int32), (0, pad)).reshape(NCH, CH)
    outd = jnp.pad(out_degree.astype(jnp.int32), (0, pad)).reshape(NCH, CH)
    wsum, idxc = pl.pallas_call(
        _prep_body,
        out_shape=[
            jax.ShapeDtypeStruct((VOCAB, VOCAB, D), jnp.float32),
            jax.ShapeDtypeStruct((NCH, CH), jnp.int32),
        ],
    )(W_in, W_out, ind, outd)
    return _sc_gather(wsum.reshape(VOCAB * VOCAB, D), idxc.reshape(PAD_N))
